# baseline (speedup 1.0000x reference)
import functools

import jax
import jax.numpy as jnp
from jax import lax
from jax.experimental import pallas as pl
from jax.experimental.pallas import tpu as pltpu
from jax.experimental.pallas import tpu_sc as plsc

NC = 2
NS = 16
NW = NC * NS
BR = 8192


def _mesh():
    return plsc.VectorSubcoreMesh(core_axis_name="c", subcore_axis_name="s",
                                  num_cores=NC, num_subcores=NS)


def _make_deg_kernel(E, NP):
    ngroups = E // 1024
    base_g, extra = divmod(ngroups, NW)
    slab = NP // NS

    @functools.partial(
        pl.kernel,
        out_type=jax.ShapeDtypeStruct((NC, NP), jnp.float32),
        mesh=_mesh(),
        scratch_types=[
            pltpu.VMEM((8, 128), jnp.int32),
            pltpu.VMEM((128,), jnp.float32),
            pltpu.VMEM_SHARED((NP,), jnp.float32),
            pltpu.SemaphoreType.DMA,
        ],
        compiler_params=pltpu.CompilerParams(use_tc_tiling_on_sc=False),
    )
    def deg_kernel(dst2d, ones_hbm, zeros_hbm, out, dbuf, onesv, hist, sem):
        c = lax.axis_index("c")
        s = lax.axis_index("s")
        wid = c * NS + s
        pltpu.sync_copy(zeros_hbm.at[pl.ds(s * slab, slab)],
                        hist.at[pl.ds(s * slab, slab)])
        pltpu.sync_copy(ones_hbm, onesv)
        plsc.subcore_barrier()

        def group(g, carry):
            gi = g * NW + wid
            pltpu.sync_copy(dst2d.at[pl.ds(gi * 8, 8), :], dbuf)
            ds = [pltpu.async_copy(onesv, hist.at[dbuf.at[j]], sem, add=True)
                  for j in range(8)]
            for d in ds:
                d.wait()
            return carry

        lax.fori_loop(0, base_g, group, 0)
        if extra:
            @pl.when(wid < extra)
            def _():
                group(base_g, 0)
        plsc.subcore_barrier()
        pltpu.sync_copy(hist.at[pl.ds(s * slab, slab)],
                        out.at[c, pl.ds(s * slab, slab)])

    return deg_kernel


def _make_mp_kernel(E, NP):
    ngroups = E // 512
    assert ngroups % NS == 0
    n_per_tec = ngroups // NS
    slab = NP // NS

    @functools.partial(
        pl.kernel,
        out_type=jax.ShapeDtypeStruct((NC, NP, 16), jnp.float32),
        mesh=_mesh(),
        scratch_types=[
            pltpu.VMEM((2, 4, 128), jnp.int32),
            pltpu.VMEM((2, 4, 128), jnp.int32),
            pltpu.VMEM((2, 4, 128, 16), jnp.float32),
            pltpu.VMEM_SHARED((NP, 16), jnp.float32),
            pltpu.SemaphoreType.DMA,
            pltpu.SemaphoreType.DMA,
            pltpu.SemaphoreType.DMA,
        ],
        compiler_params=pltpu.CompilerParams(use_tc_tiling_on_sc=False),
    )
    def mp_kernel(ya, yb, src2d, dst2d, zeros_hbm, out,
                  sbuf, dbuf, rows, acc, sem_i, sem_g, sem_s):
        c = lax.axis_index("c")
        s = lax.axis_index("s")
        pltpu.sync_copy(zeros_hbm.at[pl.ds(s * slab, slab), :],
                        acc.at[pl.ds(s * slab, slab), :])
        plsc.subcore_barrier()

        def fire_idx(g, slot):
            gi = g * NS + s
            pltpu.async_copy(src2d.at[pl.ds(gi * 4, 4), :], sbuf.at[slot], sem_i)
            pltpu.async_copy(dst2d.at[pl.ds(gi * 4, 4), :], dbuf.at[slot], sem_i)

        def drain_idx(slot):
            pltpu.make_async_copy(src2d.at[pl.ds(0, 4), :], sbuf.at[slot],
                                  sem_i).wait()
            pltpu.make_async_copy(dst2d.at[pl.ds(0, 4), :], dbuf.at[slot],
                                  sem_i).wait()

        def pipeline(tab):
            fire_idx(0, 0)
            drain_idx(0)
            for j in range(4):
                pltpu.async_copy(tab.at[sbuf.at[0, j]], rows.at[0, j], sem_g)

            def drain_scatters():
                for _k in range(4):
                    pltpu.make_async_copy(tab.at[pl.ds(0, 128), :],
                                          acc.at[pl.ds(0, 128), :],
                                          sem_s).wait()

            def group(g, carry):
                slot = lax.rem(g, 2)
                nslot = 1 - slot

                @pl.when(g > 0)
                def _():
                    drain_scatters()

                @pl.when(g < n_per_tec - 1)
                def _():
                    fire_idx(g + 1, nslot)

                for j in range(4):
                    pltpu.make_async_copy(tab.at[pl.ds(0, 128), :],
                                          rows.at[slot, j], sem_g).wait()
                    pltpu.async_copy(rows.at[slot, j],
                                     acc.at[dbuf.at[slot, j]],
                                     sem_s, add=True)

                @pl.when(g < n_per_tec - 1)
                def _():
                    drain_idx(nslot)
                    for j in range(4):
                        pltpu.async_copy(tab.at[sbuf.at[nslot, j]],
                                         rows.at[nslot, j], sem_g)
                return carry

            lax.fori_loop(0, n_per_tec, group, 0)
            drain_scatters()

        @pl.when(c == 0)
        def _():
            pipeline(ya)

        @pl.when(c == 1)
        def _():
            pipeline(yb)

        plsc.subcore_barrier()
        pltpu.sync_copy(acc.at[pl.ds(s * slab, slab), :],
                        out.at[c, pl.ds(s * slab, slab), :])

    return mp_kernel


def _stage1_body(dega, degb, x, W1, b1, Wc1, dinv_o, y1_o):
    deg = dega[...] + degb[...] + 1.0
    dinv = lax.rsqrt(deg)
    h = jnp.maximum(jnp.dot(x[...], W1[...],
                            preferred_element_type=jnp.float32) + b1[...], 0.0)
    y1_o[...] = jnp.dot(h, Wc1[...],
                        preferred_element_type=jnp.float32) * dinv[:, None]
    dinv_o[...] = dinv


def _stage_mid_body(S, y, dinv, bc, Wc, y_next_o):
    t = (S[...] + y[...]) * dinv[...][:, None] + bc[...]
    h = jnp.maximum(t, 0.0)
    y_next_o[...] = jnp.dot(h, Wc[...],
                            preferred_element_type=jnp.float32) * dinv[...][:, None]


def _stage_final_body(S, y, dinv, bc, W2, b2, W3, b3, out_o):
    t = (S[...] + y[...]) * dinv[...][:, None] + bc[...]
    h = jnp.maximum(t, 0.0)
    h = jnp.maximum(jnp.dot(h, W2[...],
                            preferred_element_type=jnp.float32) + b2[...], 0.0)
    out_o[...] = jnp.dot(h, W3[...],
                         preferred_element_type=jnp.float32) + b3[...]


def _rows_spec(F=None):
    if F is None:
        return pl.BlockSpec((BR,), lambda i: (i,))
    return pl.BlockSpec((BR, F), lambda i: (i, 0))


def _full_spec(shape):
    return pl.BlockSpec(shape, lambda i: tuple(0 for _ in shape))


def _grid(NP):
    return (pl.cdiv(NP, BR),)


def kernel(x, edge_index, W1, b1, Wc1, bc1, Wc2, bc2, Wc3, bc3, W2, b2, W3, b3):
    N = x.shape[0]
    E = edge_index.shape[1]
    F = Wc1.shape[0]
    assert E % 1024 == 0
    NP = pl.cdiv(N, 128) * 128

    GE = 512 * NS
    EP = pl.cdiv(E, GE) * GE
    src2d = edge_index[0].astype(jnp.int32).reshape(E // 128, 128)
    dst2d = edge_index[1].astype(jnp.int32).reshape(E // 128, 128)
    if EP != E:
        padrows = jnp.full(((EP - E) // 128, 128), NP - 1, jnp.int32)
        src2d = jnp.concatenate([src2d, padrows], axis=0)
        dst2d = jnp.concatenate([dst2d, padrows], axis=0)
    ones128 = jnp.ones((128,), jnp.float32)
    zeros1 = jnp.zeros((NP,), jnp.float32)
    zerosF = jnp.zeros((NP, F), jnp.float32)

    deg_k = _make_deg_kernel(EP, NP)
    mp_k = _make_mp_kernel(EP, NP)

    degp = deg_k(dst2d, ones128, zeros1)

    grid = _grid(NP)
    dinv, y1 = pl.pallas_call(
        _stage1_body,
        grid=grid,
        in_specs=[_rows_spec(), _rows_spec(), _rows_spec(2),
                  _full_spec((2, F)), _full_spec((F,)), _full_spec((F, F))],
        out_specs=[_rows_spec(), _rows_spec(F)],
        out_shape=[jax.ShapeDtypeStruct((NP,), jnp.float32),
                   jax.ShapeDtypeStruct((NP, F), jnp.float32)],
    )(degp[0], degp[1], x, W1, b1, Wc1)

    zeros16 = jnp.zeros((NP, 16), jnp.float32)

    def mp(y):
        ya = y[:, :16]
        yb = jnp.pad(y[:, 16:], ((0, 0), (0, 32 - F)))
        s = mp_k(ya, yb, src2d, dst2d, zeros16)
        return jnp.concatenate([s[0], s[1][:, :F - 16]], axis=1)

    def mid(S, y, bc, Wc):
        return pl.pallas_call(
            _stage_mid_body,
            grid=grid,
            in_specs=[_rows_spec(F), _rows_spec(F), _rows_spec(),
                      _full_spec((F,)), _full_spec((F, F))],
            out_specs=_rows_spec(F),
            out_shape=jax.ShapeDtypeStruct((NP, F), jnp.float32),
        )(S, y, dinv, bc, Wc)

    s1 = mp(y1)
    y2 = mid(s1, y1, bc1, Wc2)
    s2 = mp(y2)
    y3 = mid(s2, y2, bc2, Wc3)
    s3 = mp(y3)

    F2 = W2.shape[1]
    out = pl.pallas_call(
        _stage_final_body,
        grid=grid,
        in_specs=[_rows_spec(F), _rows_spec(F), _rows_spec(),
                  _full_spec((F,)), _full_spec((F, F2)), _full_spec((F2,)),
                  _full_spec((F2, 1)), _full_spec((1,))],
        out_specs=_rows_spec(1),
        out_shape=jax.ShapeDtypeStruct((N, 1), jnp.float32),
    )(s3, y3, dinv, bc3, W2, b2, W3, b3)
    return out


# speedup vs baseline: 1.0655x; 1.0655x over previous
import functools

import jax
import jax.numpy as jnp
from jax import lax
from jax.experimental import pallas as pl
from jax.experimental.pallas import tpu as pltpu
from jax.experimental.pallas import tpu_sc as plsc

NC = 2
NS = 16
NW = NC * NS
BR = 8192


def _mesh():
    return plsc.VectorSubcoreMesh(core_axis_name="c", subcore_axis_name="s",
                                  num_cores=NC, num_subcores=NS)


def _make_deg_kernel(E, NP):
    ngroups = E // 1024
    base_g, extra = divmod(ngroups, NW)
    slab = NP // NS

    @functools.partial(
        pl.kernel,
        out_type=jax.ShapeDtypeStruct((NC, NP), jnp.float32),
        mesh=_mesh(),
        scratch_types=[
            pltpu.VMEM((8, 128), jnp.int32),
            pltpu.VMEM((128,), jnp.float32),
            pltpu.VMEM_SHARED((NP,), jnp.float32),
            pltpu.SemaphoreType.DMA,
        ],
        compiler_params=pltpu.CompilerParams(use_tc_tiling_on_sc=False),
    )
    def deg_kernel(dst2d, ones_hbm, zeros_hbm, out, dbuf, onesv, hist, sem):
        c = lax.axis_index("c")
        s = lax.axis_index("s")
        wid = c * NS + s
        pltpu.sync_copy(zeros_hbm.at[pl.ds(s * slab, slab)],
                        hist.at[pl.ds(s * slab, slab)])
        pltpu.sync_copy(ones_hbm, onesv)
        plsc.subcore_barrier()

        def group(g, carry):
            gi = g * NW + wid
            pltpu.sync_copy(dst2d.at[pl.ds(gi * 8, 8), :], dbuf)
            ds = [pltpu.async_copy(onesv, hist.at[dbuf.at[j]], sem, add=True)
                  for j in range(8)]
            for d in ds:
                d.wait()
            return carry

        lax.fori_loop(0, base_g, group, 0)
        if extra:
            @pl.when(wid < extra)
            def _():
                group(base_g, 0)
        plsc.subcore_barrier()
        pltpu.sync_copy(hist.at[pl.ds(s * slab, slab)],
                        out.at[c, pl.ds(s * slab, slab)])

    return deg_kernel


def _make_mp_kernel(E, NP):
    ngroups = E // 768
    assert ngroups % NS == 0
    n_per_tec = ngroups // NS
    slab = NP // NS

    @functools.partial(
        pl.kernel,
        out_type=jax.ShapeDtypeStruct((NC, NP, 16), jnp.float32),
        mesh=_mesh(),
        scratch_types=[
            pltpu.VMEM((2, 6, 128), jnp.int32),
            pltpu.VMEM((2, 6, 128), jnp.int32),
            pltpu.VMEM((2, 6, 128, 16), jnp.float32),
            pltpu.VMEM_SHARED((NP, 16), jnp.float32),
            pltpu.SemaphoreType.DMA,
            pltpu.SemaphoreType.DMA,
            pltpu.SemaphoreType.DMA,
        ],
        compiler_params=pltpu.CompilerParams(use_tc_tiling_on_sc=False),
    )
    def mp_kernel(ya, yb, src2d, dst2d, zeros_hbm, out,
                  sbuf, dbuf, rows, acc, sem_i, sem_g, sem_s):
        c = lax.axis_index("c")
        s = lax.axis_index("s")
        pltpu.sync_copy(zeros_hbm.at[pl.ds(s * slab, slab), :],
                        acc.at[pl.ds(s * slab, slab), :])
        plsc.subcore_barrier()

        def fire_idx(g, slot):
            gi = g * NS + s
            pltpu.async_copy(src2d.at[pl.ds(gi * 6, 6), :], sbuf.at[slot], sem_i)
            pltpu.async_copy(dst2d.at[pl.ds(gi * 6, 6), :], dbuf.at[slot], sem_i)

        def drain_idx(slot):
            pltpu.make_async_copy(src2d.at[pl.ds(0, 6), :], sbuf.at[slot],
                                  sem_i).wait()
            pltpu.make_async_copy(dst2d.at[pl.ds(0, 6), :], dbuf.at[slot],
                                  sem_i).wait()

        def pipeline(tab):
            fire_idx(0, 0)
            drain_idx(0)
            for j in range(6):
                pltpu.async_copy(tab.at[sbuf.at[0, j]], rows.at[0, j], sem_g)

            def drain_scatters():
                for _k in range(6):
                    pltpu.make_async_copy(tab.at[pl.ds(0, 128), :],
                                          acc.at[pl.ds(0, 128), :],
                                          sem_s).wait()

            def group(g, carry):
                slot = lax.rem(g, 2)
                nslot = 1 - slot

                @pl.when(g > 0)
                def _():
                    drain_scatters()

                @pl.when(g < n_per_tec - 1)
                def _():
                    fire_idx(g + 1, nslot)

                for j in range(6):
                    pltpu.make_async_copy(tab.at[pl.ds(0, 128), :],
                                          rows.at[slot, j], sem_g).wait()
                    pltpu.async_copy(rows.at[slot, j],
                                     acc.at[dbuf.at[slot, j]],
                                     sem_s, add=True)

                @pl.when(g < n_per_tec - 1)
                def _():
                    drain_idx(nslot)
                    for j in range(6):
                        pltpu.async_copy(tab.at[sbuf.at[nslot, j]],
                                         rows.at[nslot, j], sem_g)
                return carry

            lax.fori_loop(0, n_per_tec, group, 0)
            drain_scatters()

        @pl.when(c == 0)
        def _():
            pipeline(ya)

        @pl.when(c == 1)
        def _():
            pipeline(yb)

        plsc.subcore_barrier()
        pltpu.sync_copy(acc.at[pl.ds(s * slab, slab), :],
                        out.at[c, pl.ds(s * slab, slab), :])

    return mp_kernel


def _stage1_body(dega, degb, x, W1, b1, Wc1, dinv_o, y1_o):
    deg = dega[...] + degb[...] + 1.0
    dinv = lax.rsqrt(deg)
    h = jnp.maximum(jnp.dot(x[...], W1[...],
                            preferred_element_type=jnp.float32) + b1[...], 0.0)
    y1_o[...] = jnp.dot(h, Wc1[...],
                        preferred_element_type=jnp.float32) * dinv[:, None]
    dinv_o[...] = dinv


def _stage_mid_body(S, y, dinv, bc, Wc, y_next_o):
    t = (S[...] + y[...]) * dinv[...][:, None] + bc[...]
    h = jnp.maximum(t, 0.0)
    y_next_o[...] = jnp.dot(h, Wc[...],
                            preferred_element_type=jnp.float32) * dinv[...][:, None]


def _stage_final_body(S, y, dinv, bc, W2, b2, W3, b3, out_o):
    t = (S[...] + y[...]) * dinv[...][:, None] + bc[...]
    h = jnp.maximum(t, 0.0)
    h = jnp.maximum(jnp.dot(h, W2[...],
                            preferred_element_type=jnp.float32) + b2[...], 0.0)
    out_o[...] = jnp.dot(h, W3[...],
                         preferred_element_type=jnp.float32) + b3[...]


def _rows_spec(F=None):
    if F is None:
        return pl.BlockSpec((BR,), lambda i: (i,))
    return pl.BlockSpec((BR, F), lambda i: (i, 0))


def _full_spec(shape):
    return pl.BlockSpec(shape, lambda i: tuple(0 for _ in shape))


def _grid(NP):
    return (pl.cdiv(NP, BR),)


def kernel(x, edge_index, W1, b1, Wc1, bc1, Wc2, bc2, Wc3, bc3, W2, b2, W3, b3):
    N = x.shape[0]
    E = edge_index.shape[1]
    F = Wc1.shape[0]
    assert E % 1024 == 0
    NP = pl.cdiv(N, 128) * 128

    GE = 768 * NS
    EP = pl.cdiv(E, GE) * GE
    src2d = edge_index[0].astype(jnp.int32).reshape(E // 128, 128)
    dst2d = edge_index[1].astype(jnp.int32).reshape(E // 128, 128)
    if EP != E:
        padrows = jnp.full(((EP - E) // 128, 128), NP - 1, jnp.int32)
        src2d = jnp.concatenate([src2d, padrows], axis=0)
        dst2d = jnp.concatenate([dst2d, padrows], axis=0)
    ones128 = jnp.ones((128,), jnp.float32)
    zeros1 = jnp.zeros((NP,), jnp.float32)
    zerosF = jnp.zeros((NP, F), jnp.float32)

    deg_k = _make_deg_kernel(EP, NP)
    mp_k = _make_mp_kernel(EP, NP)

    degp = deg_k(dst2d, ones128, zeros1)

    grid = _grid(NP)
    dinv, y1 = pl.pallas_call(
        _stage1_body,
        grid=grid,
        in_specs=[_rows_spec(), _rows_spec(), _rows_spec(2),
                  _full_spec((2, F)), _full_spec((F,)), _full_spec((F, F))],
        out_specs=[_rows_spec(), _rows_spec(F)],
        out_shape=[jax.ShapeDtypeStruct((NP,), jnp.float32),
                   jax.ShapeDtypeStruct((NP, F), jnp.float32)],
    )(degp[0], degp[1], x, W1, b1, Wc1)

    zeros16 = jnp.zeros((NP, 16), jnp.float32)

    def mp(y):
        ya = y[:, :16]
        yb = jnp.pad(y[:, 16:], ((0, 0), (0, 32 - F)))
        s = mp_k(ya, yb, src2d, dst2d, zeros16)
        return jnp.concatenate([s[0], s[1][:, :F - 16]], axis=1)

    def mid(S, y, bc, Wc):
        return pl.pallas_call(
            _stage_mid_body,
            grid=grid,
            in_specs=[_rows_spec(F), _rows_spec(F), _rows_spec(),
                      _full_spec((F,)), _full_spec((F, F))],
            out_specs=_rows_spec(F),
            out_shape=jax.ShapeDtypeStruct((NP, F), jnp.float32),
        )(S, y, dinv, bc, Wc)

    s1 = mp(y1)
    y2 = mid(s1, y1, bc1, Wc2)
    s2 = mp(y2)
    y3 = mid(s2, y2, bc2, Wc3)
    s3 = mp(y3)

    F2 = W2.shape[1]
    out = pl.pallas_call(
        _stage_final_body,
        grid=grid,
        in_specs=[_rows_spec(F), _rows_spec(F), _rows_spec(),
                  _full_spec((F,)), _full_spec((F, F2)), _full_spec((F2,)),
                  _full_spec((F2, 1)), _full_spec((1,))],
        out_specs=_rows_spec(1),
        out_shape=jax.ShapeDtypeStruct((N, 1), jnp.float32),
    )(s3, y3, dinv, bc3, W2, b2, W3, b3)
    return out


# speedup vs baseline: 1.1488x; 1.0781x over previous
import functools

import jax
import jax.numpy as jnp
from jax import lax
from jax.experimental import pallas as pl
from jax.experimental.pallas import tpu as pltpu
from jax.experimental.pallas import tpu_sc as plsc

NC = 2
NS = 16
NW = NC * NS
BR = 8192


def _mesh():
    return plsc.VectorSubcoreMesh(core_axis_name="c", subcore_axis_name="s",
                                  num_cores=NC, num_subcores=NS)


def _make_deg_kernel(E, NP):
    ngroups = E // 1024
    base_g, extra = divmod(ngroups, NW)
    slab = NP // NS

    @functools.partial(
        pl.kernel,
        out_type=jax.ShapeDtypeStruct((NC, NP), jnp.float32),
        mesh=_mesh(),
        scratch_types=[
            pltpu.VMEM((8, 128), jnp.int32),
            pltpu.VMEM((128,), jnp.float32),
            pltpu.VMEM_SHARED((NP,), jnp.float32),
            pltpu.SemaphoreType.DMA,
        ],
        compiler_params=pltpu.CompilerParams(use_tc_tiling_on_sc=False),
    )
    def deg_kernel(dst2d, ones_hbm, zeros_hbm, out, dbuf, onesv, hist, sem):
        c = lax.axis_index("c")
        s = lax.axis_index("s")
        wid = c * NS + s
        pltpu.sync_copy(zeros_hbm.at[pl.ds(s * slab, slab)],
                        hist.at[pl.ds(s * slab, slab)])
        pltpu.sync_copy(ones_hbm, onesv)
        plsc.subcore_barrier()

        def group(g, carry):
            gi = g * NW + wid
            pltpu.sync_copy(dst2d.at[pl.ds(gi * 8, 8), :], dbuf)
            ds = [pltpu.async_copy(onesv, hist.at[dbuf.at[j]], sem, add=True)
                  for j in range(8)]
            for d in ds:
                d.wait()
            return carry

        lax.fori_loop(0, base_g, group, 0)
        if extra:
            @pl.when(wid < extra)
            def _():
                group(base_g, 0)
        plsc.subcore_barrier()
        pltpu.sync_copy(hist.at[pl.ds(s * slab, slab)],
                        out.at[c, pl.ds(s * slab, slab)])

    return deg_kernel


def _make_mp_kernel(E, NP):
    ngroups = E // 768
    assert ngroups % NS == 0
    n_per_tec = ngroups // NS
    slab = NP // NS

    @functools.partial(
        pl.kernel,
        out_type=jax.ShapeDtypeStruct((NC, NP, 16), jnp.float32),
        mesh=_mesh(),
        scratch_types=[
            pltpu.VMEM((2, 6, 128), jnp.int32),
            pltpu.VMEM((2, 6, 128), jnp.int32),
            pltpu.VMEM((2, 6, 128, 16), jnp.float32),
            pltpu.VMEM_SHARED((NP, 16), jnp.float32),
            pltpu.SemaphoreType.DMA,
            pltpu.SemaphoreType.DMA,
            pltpu.SemaphoreType.DMA,
        ],
        compiler_params=pltpu.CompilerParams(use_tc_tiling_on_sc=False),
    )
    def mp_kernel(ya, yb, src2d, dst2d, zeros_hbm, out,
                  sbuf, dbuf, rows, acc, sem_i, sem_g, sem_s):
        c = lax.axis_index("c")
        s = lax.axis_index("s")
        pltpu.sync_copy(zeros_hbm.at[pl.ds(s * slab, slab), :],
                        acc.at[pl.ds(s * slab, slab), :])
        plsc.subcore_barrier()

        def fire_idx(g, slot):
            gi = g * NS + s
            pltpu.async_copy(src2d.at[pl.ds(gi * 6, 6), :], sbuf.at[slot], sem_i)
            pltpu.async_copy(dst2d.at[pl.ds(gi * 6, 6), :], dbuf.at[slot], sem_i)

        def drain_idx(slot):
            pltpu.make_async_copy(src2d.at[pl.ds(0, 6), :], sbuf.at[slot],
                                  sem_i).wait()
            pltpu.make_async_copy(dst2d.at[pl.ds(0, 6), :], dbuf.at[slot],
                                  sem_i).wait()

        def pipeline(tab):
            fire_idx(0, 0)
            drain_idx(0)
            for j in range(6):
                pltpu.async_copy(tab.at[sbuf.at[0, j]], rows.at[0, j], sem_g)

            def drain_scatters():
                for _k in range(6):
                    pltpu.make_async_copy(tab.at[pl.ds(0, 128), :],
                                          acc.at[pl.ds(0, 128), :],
                                          sem_s).wait()

            def group(g, carry):
                slot = lax.rem(g, 2)
                nslot = 1 - slot

                @pl.when(g > 0)
                def _():
                    drain_scatters()

                @pl.when(g < n_per_tec - 1)
                def _():
                    fire_idx(g + 1, nslot)

                for j in range(6):
                    pltpu.make_async_copy(tab.at[pl.ds(0, 128), :],
                                          rows.at[slot, j], sem_g).wait()
                    pltpu.async_copy(rows.at[slot, j],
                                     acc.at[dbuf.at[slot, j]],
                                     sem_s, add=True)

                @pl.when(g < n_per_tec - 1)
                def _():
                    drain_idx(nslot)
                    for j in range(6):
                        pltpu.async_copy(tab.at[sbuf.at[nslot, j]],
                                         rows.at[nslot, j], sem_g)
                return carry

            lax.fori_loop(0, n_per_tec, group, 0)
            drain_scatters()

        @pl.when(c == 0)
        def _():
            pipeline(ya)

        @pl.when(c == 1)
        def _():
            pipeline(yb)

        plsc.subcore_barrier()
        pltpu.sync_copy(acc.at[pl.ds(s * slab, slab), :],
                        out.at[c, pl.ds(s * slab, slab), :])

    return mp_kernel


def _split_tables(y, ya_o, yb_o):
    F = y.shape[1]
    ya_o[...] = y[:, :16]
    yb_o[...] = jnp.concatenate(
        [y[:, 16:], jnp.zeros((y.shape[0], 32 - F), jnp.float32)], axis=1)


def _assemble_t(sa, sb, ya, yb, F):
    t16 = sa[...] + ya[...]
    t4 = (sb[...] + yb[...])[:, :F - 16]
    return jnp.concatenate([t16, t4], axis=1)


def _stage1_body(dega, degb, x, W1, b1, Wc1, dinv_o, ya_o, yb_o):
    deg = dega[...] + degb[...] + 1.0
    dinv = lax.rsqrt(deg)
    h = jnp.maximum(jnp.dot(x[...], W1[...],
                            preferred_element_type=jnp.float32) + b1[...], 0.0)
    y = jnp.dot(h, Wc1[...],
                preferred_element_type=jnp.float32) * dinv[:, None]
    _split_tables(y, ya_o, yb_o)
    dinv_o[...] = dinv


def _stage_mid_body(sa, sb, ya, yb, dinv, bc, Wc, ya_o, yb_o):
    F = Wc.shape[0]
    t = _assemble_t(sa, sb, ya, yb, F) * dinv[...][:, None] + bc[...]
    h = jnp.maximum(t, 0.0)
    y = jnp.dot(h, Wc[...],
                preferred_element_type=jnp.float32) * dinv[...][:, None]
    _split_tables(y, ya_o, yb_o)


def _stage_final_body(sa, sb, ya, yb, dinv, bc, W2, b2, W3, b3, out_o):
    F = W2.shape[0]
    t = _assemble_t(sa, sb, ya, yb, F) * dinv[...][:, None] + bc[...]
    h = jnp.maximum(t, 0.0)
    h = jnp.maximum(jnp.dot(h, W2[...],
                            preferred_element_type=jnp.float32) + b2[...], 0.0)
    out_o[...] = jnp.dot(h, W3[...],
                         preferred_element_type=jnp.float32) + b3[...]


def _rows_spec(F=None):
    if F is None:
        return pl.BlockSpec((BR,), lambda i: (i,))
    return pl.BlockSpec((BR, F), lambda i: (i, 0))


def _full_spec(shape):
    return pl.BlockSpec(shape, lambda i: tuple(0 for _ in shape))


def _grid(NP):
    return (pl.cdiv(NP, BR),)


def kernel(x, edge_index, W1, b1, Wc1, bc1, Wc2, bc2, Wc3, bc3, W2, b2, W3, b3):
    N = x.shape[0]
    E = edge_index.shape[1]
    F = Wc1.shape[0]
    assert E % 1024 == 0
    NP = pl.cdiv(N, 128) * 128

    GE = 768 * NS
    EP = pl.cdiv(E, GE) * GE
    src2d = edge_index[0].astype(jnp.int32).reshape(E // 128, 128)
    dst2d = edge_index[1].astype(jnp.int32).reshape(E // 128, 128)
    if EP != E:
        padrows = jnp.full(((EP - E) // 128, 128), NP - 1, jnp.int32)
        src2d = jnp.concatenate([src2d, padrows], axis=0)
        dst2d = jnp.concatenate([dst2d, padrows], axis=0)
    ones128 = jnp.ones((128,), jnp.float32)
    zeros1 = jnp.zeros((NP,), jnp.float32)
    zerosF = jnp.zeros((NP, F), jnp.float32)

    deg_k = _make_deg_kernel(EP, NP)
    mp_k = _make_mp_kernel(EP, NP)

    degp = deg_k(dst2d, ones128, zeros1)

    grid = _grid(NP)
    dinv, ya1, yb1 = pl.pallas_call(
        _stage1_body,
        grid=grid,
        in_specs=[_rows_spec(), _rows_spec(), _rows_spec(2),
                  _full_spec((2, F)), _full_spec((F,)), _full_spec((F, F))],
        out_specs=[_rows_spec(), _rows_spec(16), _rows_spec(16)],
        out_shape=[jax.ShapeDtypeStruct((NP,), jnp.float32),
                   jax.ShapeDtypeStruct((NP, 16), jnp.float32),
                   jax.ShapeDtypeStruct((NP, 16), jnp.float32)],
    )(degp[0], degp[1], x, W1, b1, Wc1)

    zeros16 = jnp.zeros((NP, 16), jnp.float32)

    def mid(sp, ya, yb, bc, Wc):
        return pl.pallas_call(
            _stage_mid_body,
            grid=grid,
            in_specs=[_rows_spec(16), _rows_spec(16), _rows_spec(16),
                      _rows_spec(16), _rows_spec(),
                      _full_spec((F,)), _full_spec((F, F))],
            out_specs=[_rows_spec(16), _rows_spec(16)],
            out_shape=[jax.ShapeDtypeStruct((NP, 16), jnp.float32),
                       jax.ShapeDtypeStruct((NP, 16), jnp.float32)],
        )(sp[0], sp[1], ya, yb, dinv, bc, Wc)

    s1 = mp_k(ya1, yb1, src2d, dst2d, zeros16)
    ya2, yb2 = mid(s1, ya1, yb1, bc1, Wc2)
    s2 = mp_k(ya2, yb2, src2d, dst2d, zeros16)
    ya3, yb3 = mid(s2, ya2, yb2, bc2, Wc3)
    s3 = mp_k(ya3, yb3, src2d, dst2d, zeros16)

    F2 = W2.shape[1]
    out = pl.pallas_call(
        _stage_final_body,
        grid=grid,
        in_specs=[_rows_spec(16), _rows_spec(16), _rows_spec(16),
                  _rows_spec(16), _rows_spec(),
                  _full_spec((F,)), _full_spec((F, F2)), _full_spec((F2,)),
                  _full_spec((F2, 1)), _full_spec((1,))],
        out_specs=_rows_spec(1),
        out_shape=jax.ShapeDtypeStruct((N, 1), jnp.float32),
    )(s3[0], s3[1], ya3, yb3, dinv, bc3, W2, b2, W3, b3)
    return out
